# baseline (speedup 1.0000x reference)
import functools

import jax
import jax.numpy as jnp
from jax import lax
from jax.experimental import pallas as pl
from jax.experimental.pallas import tpu as pltpu
from jax.experimental.pallas import tpu_sc as plsc

_NC = 2
_NS = 16


def _sc_build_res(hist, user, item, cate, u_table, i_table, c_table, iota,
                  zeros):
    L, B = hist.shape
    D = i_table.shape[1]
    bpw = B // (_NC * _NS)

    mesh = plsc.VectorSubcoreMesh(core_axis_name="c", subcore_axis_name="s")

    @functools.partial(
        pl.kernel,
        mesh=mesh,
        out_type=jax.ShapeDtypeStruct((B, 4 * D), jnp.float32),
        scratch_types=[
            pltpu.VMEM((L, bpw), jnp.int32),
            pltpu.VMEM((bpw,), jnp.int32),
            pltpu.VMEM((bpw,), jnp.int32),
            pltpu.VMEM((bpw, D), jnp.float32),
            pltpu.VMEM((bpw, D), jnp.float32),
            pltpu.VMEM((bpw, D), jnp.float32),
            pltpu.VMEM_SHARED((_NS * bpw, D), jnp.float32),
            pltpu.SemaphoreType.DMA,
            pltpu.SemaphoreType.DMA,
        ],
    )
    def k(hist_hbm, user_hbm, item_hbm, cate_hbm, ut_hbm, it_hbm, ct_hbm,
          iota_hbm, zeros_hbm, res_hbm, hist_v, idx_v, qidx_v, buf0, buf1,
          ubuf, acc_sh, sem0, sem1):
        c = lax.axis_index("c")
        s = lax.axis_index("s")
        gbase = (c * _NS + s) * bpw

        pltpu.sync_copy(iota_hbm.at[s], idx_v)
        pltpu.sync_copy(hist_hbm.at[:, pl.ds(gbase, bpw)], hist_v)

        pltpu.async_copy(it_hbm.at[hist_v.at[0]], buf0, sem0)
        pltpu.sync_copy(zeros_hbm, acc_sh.at[pl.ds(s * bpw, bpw)])

        pltpu.sync_copy(user_hbm.at[pl.ds(gbase, bpw)], qidx_v)
        pltpu.sync_copy(ut_hbm.at[qidx_v], ubuf)
        pltpu.sync_copy(ubuf, res_hbm.at[pl.ds(gbase, bpw), pl.ds(0, D)])

        pltpu.sync_copy(item_hbm.at[pl.ds(gbase, bpw)], qidx_v)
        pltpu.sync_copy(it_hbm.at[qidx_v], ubuf)
        pltpu.sync_copy(ubuf, res_hbm.at[pl.ds(gbase, bpw), pl.ds(D, D)])

        pltpu.sync_copy(cate_hbm.at[pl.ds(gbase, bpw)], qidx_v)
        pltpu.sync_copy(ct_hbm.at[qidx_v], ubuf)
        pltpu.sync_copy(ubuf, res_hbm.at[pl.ds(gbase, bpw), pl.ds(2 * D, D)])

        pltpu.make_async_copy(it_hbm.at[hist_v.at[0]], buf0, sem0).wait()
        pltpu.sync_copy(it_hbm.at[hist_v.at[1]], buf1)

        @pl.loop(2, L, step=2)
        def _(l):
            pltpu.sync_copy(it_hbm.at[hist_v.at[l]], buf0, add=True)
            pltpu.sync_copy(it_hbm.at[hist_v.at[l + 1]], buf1, add=True)

        pltpu.sync_copy(buf0, acc_sh.at[idx_v])
        pltpu.sync_copy(buf1, acc_sh.at[idx_v], add=True)
        pltpu.sync_copy(acc_sh.at[pl.ds(s * bpw, bpw)],
                        res_hbm.at[pl.ds(gbase, bpw), pl.ds(3 * D, D)])

    return k(hist, user, item, cate, u_table, i_table, c_table, iota, zeros)


def _dice(x, alpha):
    mean = jnp.mean(x, axis=0, keepdims=True)
    var = jnp.mean((x - mean) ** 2, axis=0, keepdims=True)
    x_n = (x - mean) * lax.rsqrt(var + 1e-8)
    p = jax.nn.sigmoid(x_n)
    return p * x + (1.0 - p) * alpha * x


def _mlp_body(res_ref, w1_ref, b1_ref, a1_ref, w2_ref, b2_ref, a2_ref,
              w3_ref, b3_ref, out_ref):
    x = res_ref[...]
    h = jnp.dot(x, w1_ref[...], preferred_element_type=jnp.float32) + b1_ref[...]
    h = _dice(h, a1_ref[...])
    h = jnp.dot(h, w2_ref[...], preferred_element_type=jnp.float32) + b2_ref[...]
    h = _dice(h, a2_ref[...])
    out_ref[...] = (
        jnp.dot(h, w3_ref[...], preferred_element_type=jnp.float32) + b3_ref[...]
    )


def kernel(user, hist, item, cate, u_table, i_table, c_table,
           W1, b1, alpha1, W2, b2, alpha2, W3, b3):
    B = user.shape[0]
    bpw = B // (_NC * _NS)
    iota = (jnp.arange(_NS, dtype=jnp.int32)[:, None] * bpw
            + jnp.arange(bpw, dtype=jnp.int32)[None, :])
    zeros = jnp.zeros((bpw, i_table.shape[1]), jnp.float32)

    res = _sc_build_res(hist, user, item, cate, u_table, i_table, c_table,
                        iota, zeros)

    n_out = W3.shape[1]
    w3p = jnp.zeros((W3.shape[0], 128), jnp.float32).at[:, :n_out].set(W3)
    b3p = jnp.zeros((128,), jnp.float32).at[:n_out].set(b3)

    out = pl.pallas_call(
        _mlp_body,
        out_shape=jax.ShapeDtypeStruct((B, 128), jnp.float32),
    )(res, W1, b1.reshape(1, -1), alpha1.reshape(1, -1),
      W2, b2.reshape(1, -1), alpha2.reshape(1, -1),
      w3p, b3p.reshape(1, -1))
    return out[:, :n_out]


# speedup vs baseline: 1.6662x; 1.6662x over previous
import functools

import jax
import jax.numpy as jnp
from jax import lax
from jax.experimental import pallas as pl
from jax.experimental.pallas import tpu as pltpu
from jax.experimental.pallas import tpu_sc as plsc

_NC = 2
_NS = 16
_NBUF = 6


def _sc_build_res(hist, user, item, cate, u_table, i_table, c_table):
    L, B = hist.shape
    D = i_table.shape[1]
    bpw = B // (_NC * _NS)
    nbuf = _NBUF
    tail = (L - nbuf) % nbuf

    mesh = plsc.VectorSubcoreMesh(core_axis_name="c", subcore_axis_name="s")

    @functools.partial(
        pl.kernel,
        mesh=mesh,
        out_type=jax.ShapeDtypeStruct((B, (3 + nbuf) * D), jnp.float32),
        scratch_types=[
            pltpu.VMEM((L, bpw), jnp.int32),
            pltpu.VMEM((bpw,), jnp.int32),
        ] + [pltpu.VMEM((bpw, D), jnp.float32) for _ in range(nbuf)]
          + [pltpu.SemaphoreType.DMA for _ in range(nbuf)],
    )
    def k(hist_hbm, user_hbm, item_hbm, cate_hbm, ut_hbm, it_hbm, ct_hbm,
          res_hbm, hist_v, qidx_v, *bufsems):
        bufs, sems = bufsems[:nbuf], bufsems[nbuf:]
        c = lax.axis_index("c")
        s = lax.axis_index("s")
        gbase = (c * _NS + s) * bpw

        pltpu.sync_copy(hist_hbm.at[:, pl.ds(gbase, bpw)], hist_v)

        for j in range(1, nbuf):
            pltpu.async_copy(it_hbm.at[hist_v.at[j]], bufs[j], sems[j])

        pltpu.sync_copy(user_hbm.at[pl.ds(gbase, bpw)], qidx_v)
        pltpu.sync_copy(ut_hbm.at[qidx_v], bufs[0])
        pltpu.sync_copy(bufs[0], res_hbm.at[pl.ds(gbase, bpw), pl.ds(0, D)])

        pltpu.sync_copy(item_hbm.at[pl.ds(gbase, bpw)], qidx_v)
        pltpu.sync_copy(it_hbm.at[qidx_v], bufs[0])
        pltpu.sync_copy(bufs[0], res_hbm.at[pl.ds(gbase, bpw), pl.ds(D, D)])

        pltpu.sync_copy(cate_hbm.at[pl.ds(gbase, bpw)], qidx_v)
        pltpu.sync_copy(ct_hbm.at[qidx_v], bufs[0])
        pltpu.sync_copy(bufs[0], res_hbm.at[pl.ds(gbase, bpw), pl.ds(2 * D, D)])

        pltpu.async_copy(it_hbm.at[hist_v.at[0]], bufs[0], sems[0])

        @pl.loop(nbuf, L - tail, step=nbuf)
        def _(l):
            for j in range(nbuf):
                pltpu.make_async_copy(
                    it_hbm.at[hist_v.at[l + j]], bufs[j], sems[j]).wait()
                pltpu.async_copy(
                    it_hbm.at[hist_v.at[l + j]], bufs[j], sems[j], add=True)

        for j in range(tail):
            pltpu.make_async_copy(it_hbm.at[hist_v.at[j]], bufs[j], sems[j]).wait()
            pltpu.async_copy(
                it_hbm.at[hist_v.at[L - tail + j]], bufs[j], sems[j], add=True)

        for j in range(nbuf):
            pltpu.make_async_copy(it_hbm.at[hist_v.at[j]], bufs[j], sems[j]).wait()
            pltpu.sync_copy(bufs[j],
                            res_hbm.at[pl.ds(gbase, bpw), pl.ds((3 + j) * D, D)])

    return k(hist, user, item, cate, u_table, i_table, c_table)


def _dice(x, alpha):
    mean = jnp.mean(x, axis=0, keepdims=True)
    var = jnp.mean((x - mean) ** 2, axis=0, keepdims=True)
    x_n = (x - mean) * lax.rsqrt(var + 1e-8)
    p = jax.nn.sigmoid(x_n)
    return p * x + (1.0 - p) * alpha * x


def _mlp_body(res_ref, w1_ref, b1_ref, a1_ref, w2_ref, b2_ref, a2_ref,
              w3_ref, b3_ref, out_ref):
    r = res_ref[...]
    d = w1_ref.shape[0] // 4
    cur = r[:, 3 * d:4 * d]
    for j in range(1, (r.shape[1] - 3 * d) // d):
        cur = cur + r[:, (3 + j) * d:(4 + j) * d]
    x = jnp.concatenate([r[:, :3 * d], cur], axis=1)
    h = jnp.dot(x, w1_ref[...], preferred_element_type=jnp.float32) + b1_ref[...]
    h = _dice(h, a1_ref[...])
    h = jnp.dot(h, w2_ref[...], preferred_element_type=jnp.float32) + b2_ref[...]
    h = _dice(h, a2_ref[...])
    out_ref[...] = (
        jnp.dot(h, w3_ref[...], preferred_element_type=jnp.float32) + b3_ref[...]
    )


def kernel(user, hist, item, cate, u_table, i_table, c_table,
           W1, b1, alpha1, W2, b2, alpha2, W3, b3):
    B = user.shape[0]
    D = i_table.shape[1]

    res = _sc_build_res(hist, user, item, cate, u_table, i_table, c_table)

    n_out = W3.shape[1]
    w3p = jnp.zeros((W3.shape[0], 128), jnp.float32).at[:, :n_out].set(W3)
    b3p = jnp.zeros((128,), jnp.float32).at[:n_out].set(b3)

    out = pl.pallas_call(
        _mlp_body,
        out_shape=jax.ShapeDtypeStruct((B, 128), jnp.float32),
    )(res, W1, b1.reshape(1, -1), alpha1.reshape(1, -1),
      W2, b2.reshape(1, -1), alpha2.reshape(1, -1),
      w3p, b3p.reshape(1, -1))
    return out[:, :n_out]
